# baseline (speedup 1.0000x reference)
import jax
import jax.numpy as jnp
from jax.experimental import pallas as pl
from jax.experimental.pallas import tpu as pltpu


def _blend_body(ts_ref, sac_ref, somac_ref, x0_ref, noise_ref, out_ref):
    b = pl.program_id(0)
    t = ts_ref[b]
    sac = sac_ref[t]
    somac = somac_ref[t]
    out_ref[...] = sac * x0_ref[...] + somac * noise_ref[...]


def kernel(x0, noise, time_steps, sqrt_alpha_cumprod, sqrt_one_minus_alpha_cumprod):
    B, C, H, W = x0.shape
    ROWS = C * H * W // 128
    x3 = x0.reshape(B, ROWS, 128)
    n3 = noise.reshape(B, ROWS, 128)
    ts = time_steps.astype(jnp.int32)

    out = pl.pallas_call(
        _blend_body,
        grid=(B,),
        in_specs=[
            pl.BlockSpec(memory_space=pltpu.SMEM),
            pl.BlockSpec(memory_space=pltpu.SMEM),
            pl.BlockSpec(memory_space=pltpu.SMEM),
            pl.BlockSpec((1, ROWS, 128), lambda b: (b, 0, 0)),
            pl.BlockSpec((1, ROWS, 128), lambda b: (b, 0, 0)),
        ],
        out_specs=pl.BlockSpec((1, ROWS, 128), lambda b: (b, 0, 0)),
        out_shape=jax.ShapeDtypeStruct((B, ROWS, 128), jnp.float32),
    )(ts, sqrt_alpha_cumprod, sqrt_one_minus_alpha_cumprod, x3, n3)
    return out.reshape(B, C, H, W)


# speedup vs baseline: 1.1015x; 1.1015x over previous
import jax
import jax.numpy as jnp
from jax.experimental import pallas as pl
from jax.experimental.pallas import tpu as pltpu


_BLK_B = 8


def _blend_body(ts_ref, sac_ref, somac_ref, x0_ref, noise_ref, out_ref):
    i = pl.program_id(0)
    for j in range(_BLK_B):
        t = ts_ref[i * _BLK_B + j]
        out_ref[j] = sac_ref[t] * x0_ref[j] + somac_ref[t] * noise_ref[j]


def kernel(x0, noise, time_steps, sqrt_alpha_cumprod, sqrt_one_minus_alpha_cumprod):
    B, C, H, W = x0.shape
    ROWS = C * H * W // 128
    x3 = x0.reshape(B, ROWS, 128)
    n3 = noise.reshape(B, ROWS, 128)
    ts = time_steps.astype(jnp.int32)

    out = pl.pallas_call(
        _blend_body,
        grid=(B // _BLK_B,),
        in_specs=[
            pl.BlockSpec(memory_space=pltpu.SMEM),
            pl.BlockSpec(memory_space=pltpu.SMEM),
            pl.BlockSpec(memory_space=pltpu.SMEM),
            pl.BlockSpec((_BLK_B, ROWS, 128), lambda b: (b, 0, 0)),
            pl.BlockSpec((_BLK_B, ROWS, 128), lambda b: (b, 0, 0)),
        ],
        out_specs=pl.BlockSpec((_BLK_B, ROWS, 128), lambda b: (b, 0, 0)),
        out_shape=jax.ShapeDtypeStruct((B, ROWS, 128), jnp.float32),
    )(ts, sqrt_alpha_cumprod, sqrt_one_minus_alpha_cumprod, x3, n3)
    return out.reshape(B, C, H, W)


# speedup vs baseline: 1.1020x; 1.0005x over previous
import jax
import jax.numpy as jnp
from jax.experimental import pallas as pl
from jax.experimental.pallas import tpu as pltpu

_KS = 4
_NBUF = 4


def _blend_body(ts_ref, sac_ref, somac_ref, x0_hbm, noise_hbm, out_hbm,
                bufx, bufn, bufo, insem, outsem):
    B = x0_hbm.shape[0]
    nchunk = B // _KS

    def in_copies(c, slot):
        cx = pltpu.make_async_copy(
            x0_hbm.at[pl.ds(c * _KS, _KS)], bufx.at[slot], insem.at[slot, 0])
        cn = pltpu.make_async_copy(
            noise_hbm.at[pl.ds(c * _KS, _KS)], bufn.at[slot], insem.at[slot, 1])
        return cx, cn

    def out_copy(c, slot):
        return pltpu.make_async_copy(
            bufo.at[slot], out_hbm.at[pl.ds(c * _KS, _KS)], outsem.at[slot])

    for c in range(_NBUF):
        cx, cn = in_copies(c, c)
        cx.start()
        cn.start()

    for c in range(nchunk):
        slot = c % _NBUF
        cx, cn = in_copies(c, slot)
        cx.wait()
        cn.wait()
        if c >= _NBUF:
            out_copy(c - _NBUF, slot).wait()
        for j in range(_KS):
            t = ts_ref[c * _KS + j]
            bufo[slot, j] = sac_ref[t] * bufx[slot, j] + somac_ref[t] * bufn[slot, j]
        out_copy(c, slot).start()
        if c + _NBUF < nchunk:
            cx, cn = in_copies(c + _NBUF, slot)
            cx.start()
            cn.start()

    for c in range(nchunk - _NBUF, nchunk):
        out_copy(c, c % _NBUF).wait()


def kernel(x0, noise, time_steps, sqrt_alpha_cumprod, sqrt_one_minus_alpha_cumprod):
    B, C, H, W = x0.shape
    ROWS = C * H * W // 128
    x3 = x0.reshape(B, ROWS, 128)
    n3 = noise.reshape(B, ROWS, 128)
    ts = time_steps.astype(jnp.int32)

    out = pl.pallas_call(
        _blend_body,
        in_specs=[
            pl.BlockSpec(memory_space=pltpu.SMEM),
            pl.BlockSpec(memory_space=pltpu.SMEM),
            pl.BlockSpec(memory_space=pltpu.SMEM),
            pl.BlockSpec(memory_space=pltpu.HBM),
            pl.BlockSpec(memory_space=pltpu.HBM),
        ],
        out_specs=pl.BlockSpec(memory_space=pltpu.HBM),
        out_shape=jax.ShapeDtypeStruct((B, ROWS, 128), jnp.float32),
        scratch_shapes=[
            pltpu.VMEM((_NBUF, _KS, ROWS, 128), jnp.float32),
            pltpu.VMEM((_NBUF, _KS, ROWS, 128), jnp.float32),
            pltpu.VMEM((_NBUF, _KS, ROWS, 128), jnp.float32),
            pltpu.SemaphoreType.DMA((_NBUF, 2)),
            pltpu.SemaphoreType.DMA((_NBUF,)),
        ],
    )(ts, sqrt_alpha_cumprod, sqrt_one_minus_alpha_cumprod, x3, n3)
    return out.reshape(B, C, H, W)


# speedup vs baseline: 4.6726x; 4.2400x over previous
import jax
import jax.numpy as jnp
from jax.experimental import pallas as pl
from jax.experimental.pallas import tpu as pltpu

_BLK_B = 8


def _blend_body(ts_ref, sac_ref, somac_ref, x0_ref, noise_ref, out_ref):
    i = pl.program_id(0)
    for j in range(_BLK_B):
        t = ts_ref[i * _BLK_B + j]
        out_ref[j] = sac_ref[t] * x0_ref[j] + somac_ref[t] * noise_ref[j]


def kernel(x0, noise, time_steps, sqrt_alpha_cumprod, sqrt_one_minus_alpha_cumprod):
    B, C, H, W = x0.shape
    ts = time_steps.astype(jnp.int32)

    out = pl.pallas_call(
        _blend_body,
        grid=(B // _BLK_B,),
        in_specs=[
            pl.BlockSpec(memory_space=pltpu.SMEM),
            pl.BlockSpec(memory_space=pltpu.SMEM),
            pl.BlockSpec(memory_space=pltpu.SMEM),
            pl.BlockSpec((_BLK_B, C, H, W), lambda b: (b, 0, 0, 0)),
            pl.BlockSpec((_BLK_B, C, H, W), lambda b: (b, 0, 0, 0)),
        ],
        out_specs=pl.BlockSpec((_BLK_B, C, H, W), lambda b: (b, 0, 0, 0)),
        out_shape=jax.ShapeDtypeStruct((B, C, H, W), jnp.float32),
    )(ts, sqrt_alpha_cumprod, sqrt_one_minus_alpha_cumprod, x0, noise)
    return out
